# Initial kernel scaffold; baseline (speedup 1.0000x reference)
import jax
import jax.numpy as jnp
from jax.experimental import pallas as pl


def kernel(x, edge_index, edge_attr, W_msg, W_edge, W_upd, W_self, b_msg, b_upd):
    raise NotImplementedError("write your pallas kernel here")


# speedup vs baseline: 3.0807x; 3.0807x over previous
import functools

import jax
import jax.numpy as jnp
from jax import lax
from jax.experimental import pallas as pl
from jax.experimental.pallas import tpu as pltpu
from jax.experimental.pallas import tpu_sc as plsc

N = 10000
E = 320000
D = 128
DE = 16

NC = 2
NS = 16
NW = NC * NS
EPW = E // NW
CHUNK = 80
NCHUNKS = EPW // CHUNK
RPT = 624
ZROWS = 104
TAIL = N - NS * RPT
VPR = D // 16


def _sc_edge_agg(hW, eW, snd, rcv):
    mesh = plsc.VectorSubcoreMesh(core_axis_name="c", subcore_axis_name="s")

    @functools.partial(
        pl.kernel,
        out_type=jax.ShapeDtypeStruct((NC, N, D), jnp.float32),
        mesh=mesh,
        scratch_types=[
            pltpu.VMEM((CHUNK,), jnp.int32),
            pltpu.VMEM((CHUNK,), jnp.int32),
            pltpu.VMEM((CHUNK, D), jnp.float32),
            pltpu.VMEM((CHUNK, D), jnp.float32),
            pltpu.VMEM((ZROWS, D), jnp.float32),
            pltpu.VMEM_SHARED((N, D), jnp.float32),
            pltpu.SemaphoreType.DMA,
            pltpu.SemaphoreType.DMA,
        ],
    )
    def k(hW_hbm, eW_hbm, snd_hbm, rcv_hbm, out_hbm,
          sidx, ridx, grows, erows, zbuf, agg, sem_g, sem_e):
        c = lax.axis_index("c")
        s = lax.axis_index("s")
        wid = c * NS + s

        zero = jnp.zeros((16,), jnp.float32)

        def zset(i, carry):
            for w in range(VPR):
                zbuf[i, pl.ds(w * 16, 16)] = zero
            return carry

        lax.fori_loop(0, ZROWS, zset, 0)
        for j in range(RPT // ZROWS):
            pltpu.sync_copy(zbuf, agg.at[pl.ds(s * RPT + j * ZROWS, ZROWS)])

        @pl.when(s == NS - 1)
        def _():
            pltpu.sync_copy(zbuf.at[pl.ds(0, TAIL)],
                            agg.at[pl.ds(NS * RPT, TAIL)])

        plsc.subcore_barrier()

        base0 = wid * EPW

        def chunk_body(ci, carry):
            base = base0 + ci * CHUNK
            pltpu.sync_copy(snd_hbm.at[pl.ds(base, CHUNK)], sidx)
            pltpu.sync_copy(rcv_hbm.at[pl.ds(base, CHUNK)], ridx)
            cg = pltpu.async_copy(hW_hbm.at[sidx], grows, sem_g)
            ce = pltpu.async_copy(eW_hbm.at[pl.ds(base, CHUNK)], erows, sem_e)
            ce.wait()
            cg.wait()

            def ebody(e, ecarry):
                for w in range(VPR):
                    sl = pl.ds(w * 16, 16)
                    grows[e, sl] = jnp.maximum(grows[e, sl] + erows[e, sl], 0.0)
                return ecarry

            lax.fori_loop(0, CHUNK, ebody, 0)
            pltpu.sync_copy(grows, agg.at[ridx], add=True)
            return carry

        lax.fori_loop(0, NCHUNKS, chunk_body, 0)
        plsc.subcore_barrier()

        for j in range(RPT // ZROWS):
            off = s * RPT + j * ZROWS
            pltpu.sync_copy(agg.at[pl.ds(off, ZROWS)],
                            out_hbm.at[c, pl.ds(off, ZROWS)])

        @pl.when(s == NS - 1)
        def _():
            pltpu.sync_copy(agg.at[pl.ds(NS * RPT, TAIL)],
                            out_hbm.at[c, pl.ds(NS * RPT, TAIL)])

    return k(hW, eW, snd, rcv)


_NBLK = 1000
_EBLK = 4000


def _node_proj_body(h_ref, w_ref, b_ref, o_ref):
    o_ref[...] = jnp.dot(h_ref[...], w_ref[...],
                         preferred_element_type=jnp.float32) + b_ref[...]


def _node_proj(h, w, b):
    return pl.pallas_call(
        _node_proj_body,
        grid=(N // _NBLK,),
        in_specs=[
            pl.BlockSpec((_NBLK, D), lambda i: (i, 0)),
            pl.BlockSpec((D, D), lambda i: (0, 0)),
            pl.BlockSpec((1, D), lambda i: (0, 0)),
        ],
        out_specs=pl.BlockSpec((_NBLK, D), lambda i: (i, 0)),
        out_shape=jax.ShapeDtypeStruct((N, D), jnp.float32),
    )(h, w, b.reshape(1, D))


def _edge_proj_body(a_ref, w_ref, o_ref):
    o_ref[...] = jnp.dot(a_ref[...], w_ref[...],
                         preferred_element_type=jnp.float32)


def _edge_proj(ea, w):
    return pl.pallas_call(
        _edge_proj_body,
        grid=(E // _EBLK,),
        in_specs=[
            pl.BlockSpec((_EBLK, DE), lambda i: (i, 0)),
            pl.BlockSpec((DE, D), lambda i: (0, 0)),
        ],
        out_specs=pl.BlockSpec((_EBLK, D), lambda i: (i, 0)),
        out_shape=jax.ShapeDtypeStruct((E, D), jnp.float32),
    )(ea, w)


def _update_body(p_ref, h_ref, wu_ref, ws_ref, b_ref, o_ref):
    agg = p_ref[0] + p_ref[1]
    t = (jnp.dot(agg, wu_ref[...], preferred_element_type=jnp.float32)
         + jnp.dot(h_ref[...], ws_ref[...], preferred_element_type=jnp.float32)
         + b_ref[...])
    o_ref[...] = h_ref[...] + jnp.maximum(t, 0.0)


def _update(parts, h, wu, ws, b):
    return pl.pallas_call(
        _update_body,
        grid=(N // _NBLK,),
        in_specs=[
            pl.BlockSpec((NC, _NBLK, D), lambda i: (0, i, 0)),
            pl.BlockSpec((_NBLK, D), lambda i: (i, 0)),
            pl.BlockSpec((D, D), lambda i: (0, 0)),
            pl.BlockSpec((D, D), lambda i: (0, 0)),
            pl.BlockSpec((1, D), lambda i: (0, 0)),
        ],
        out_specs=pl.BlockSpec((_NBLK, D), lambda i: (i, 0)),
        out_shape=jax.ShapeDtypeStruct((N, D), jnp.float32),
    )(parts, h, wu, ws, b.reshape(1, D))


def kernel(x, edge_index, edge_attr, W_msg, W_edge, W_upd, W_self, b_msg, b_upd):
    snd = edge_index[0]
    rcv = edge_index[1]
    h = x
    n_layers = W_msg.shape[0]
    for l in range(n_layers):
        eW = _edge_proj(edge_attr, W_edge[l])
        hW = _node_proj(h, W_msg[l], b_msg[l])
        parts = _sc_edge_agg(hW, eW, snd, rcv)
        h = _update(parts, h, W_upd[l], W_self[l], b_upd[l])
    return h
